# baseline (speedup 1.0000x reference)
import functools
import math

import jax
from jax import lax
import jax.numpy as jnp
from jax.experimental import pallas as pl
from jax.experimental.pallas import tpu as pltpu
from jax.experimental.pallas import tpu_sc as plsc

D_MODEL = 512
D_FF = 1024
N_HEADS = 8
N_EXPERTS = 6
HEAD_DIM = D_MODEL // N_HEADS

N_TOK = 2400
N_ASSIGN = 2 * N_TOK
RB = 256
NBLK = (N_ASSIGN + N_EXPERTS * (RB - 1)) // RB
NSLOT = NBLK * RB
SC_CHUNK = 160
SC_WORKERS = N_ASSIGN // SC_CHUNK


def _mm_kernel(x_ref, w_ref, b_ref, o_ref):
    o_ref[...] = (
        jnp.dot(x_ref[...], w_ref[...], preferred_element_type=jnp.float32)
        + b_ref[...]
    )


def _mm(x, w, b):
    n, k = x.shape
    m = w.shape[1]
    return pl.pallas_call(
        _mm_kernel,
        out_shape=jax.ShapeDtypeStruct((n, m), jnp.float32),
    )(x, w, b.reshape(1, m))


def _ln_op(y, g, b):
    mu = jnp.mean(y, axis=-1, keepdims=True)
    var = jnp.mean((y - mu) ** 2, axis=-1, keepdims=True)
    return (y - mu) * jax.lax.rsqrt(var + 1e-5) * g + b


def _mm_res_ln_kernel(x_ref, w_ref, b_ref, res_ref, g_ref, gb_ref, o_ref):
    y = (
        jnp.dot(x_ref[...], w_ref[...], preferred_element_type=jnp.float32)
        + b_ref[...]
    )
    y = y + res_ref[...]
    o_ref[...] = _ln_op(y, g_ref[...], gb_ref[...])


def _mm_res_ln(x, w, b, res, g, gb):
    n, k = x.shape
    m = w.shape[1]
    return pl.pallas_call(
        _mm_res_ln_kernel,
        out_shape=jax.ShapeDtypeStruct((n, m), jnp.float32),
    )(x, w, b.reshape(1, m), res, g.reshape(1, m), gb.reshape(1, m))


def _ln_kernel(x_ref, g_ref, b_ref, o_ref):
    o_ref[...] = _ln_op(x_ref[...], g_ref[...], b_ref[...])


def _ln(x, g, b):
    n, d = x.shape
    return pl.pallas_call(
        _ln_kernel,
        out_shape=jax.ShapeDtypeStruct((n, d), jnp.float32),
    )(x, g.reshape(1, d), b.reshape(1, d))


def _attn_kernel(q_ref, k_ref, v_ref, m_ref, o_ref):
    q = q_ref[0, 0]
    k = k_ref[0, 0]
    v = v_ref[0, 0]
    s = jax.lax.dot_general(
        q, k, (((1,), (1,)), ((), ())), preferred_element_type=jnp.float32
    ) * (1.0 / math.sqrt(HEAD_DIM))
    s = s + m_ref[...]
    mx = jnp.max(s, axis=-1, keepdims=True)
    e = jnp.exp(s - mx)
    a = e / jnp.sum(e, axis=-1, keepdims=True)
    o_ref[0, 0] = jnp.dot(a, v, preferred_element_type=jnp.float32)


def _attn_core(qh, kh, vh, mask):
    bb, h, tq, hd = qh.shape
    tk = kh.shape[2]
    return pl.pallas_call(
        _attn_kernel,
        grid=(bb, h),
        in_specs=[
            pl.BlockSpec((1, 1, tq, hd), lambda i, j: (i, j, 0, 0)),
            pl.BlockSpec((1, 1, tk, hd), lambda i, j: (i, j, 0, 0)),
            pl.BlockSpec((1, 1, tk, hd), lambda i, j: (i, j, 0, 0)),
            pl.BlockSpec((tq, tk), lambda i, j: (0, 0)),
        ],
        out_specs=pl.BlockSpec((1, 1, tq, hd), lambda i, j: (i, j, 0, 0)),
        out_shape=jax.ShapeDtypeStruct((bb, h, tq, hd), jnp.float32),
    )(qh, kh, vh, mask)


def _split_heads(y, t, b):
    return y.reshape(t, b, N_HEADS, HEAD_DIM).transpose(1, 2, 0, 3)


def _merge_heads(o, t, b):
    return o.transpose(2, 0, 1, 3).reshape(t * b, D_MODEL)


def _mha(xq, xkv, p, mask, tq, tk, b):
    wq, wk, wv = jnp.split(p["Wqkv"], 3, axis=1)
    bq, bk, bv = jnp.split(p["bqkv"], 3)
    if xq is xkv:
        qkv = _mm(xq, p["Wqkv"], p["bqkv"])
        q, k, v = jnp.split(qkv, 3, axis=1)
    else:
        q = _mm(xq, wq, bq)
        kv = _mm(xkv, jnp.concatenate([wk, wv], axis=1), jnp.concatenate([bk, bv]))
        k, v = jnp.split(kv, 2, axis=1)
    qh = _split_heads(q, tq, b)
    kh = _split_heads(k, tk, b)
    vh = _split_heads(v, tk, b)
    o = _attn_core(qh, kh, vh, mask)
    return _merge_heads(o, tq, b)


def _router_kernel(x_ref, wr_ref, slots_ref, w1_ref, w2_ref, be_ref):
    logits = jnp.dot(x_ref[...], wr_ref[...], preferred_element_type=jnp.float32)
    mx = jnp.max(logits, axis=-1, keepdims=True)
    ex = jnp.exp(logits - mx)
    p = ex / jnp.sum(ex, axis=-1, keepdims=True)
    ids6 = lax.broadcasted_iota(jnp.int32, p.shape, 1)
    m1 = jnp.max(p, axis=-1, keepdims=True)
    i1 = jnp.min(jnp.where(p == m1, ids6, N_EXPERTS), axis=-1, keepdims=True)
    p2 = jnp.where(ids6 == i1, -jnp.inf, p)
    m2 = jnp.max(p2, axis=-1, keepdims=True)
    i2 = jnp.min(jnp.where(p2 == m2, ids6, N_EXPERTS), axis=-1, keepdims=True)
    den = m1 + m2
    w1_ref[...] = m1 / den
    w2_ref[...] = m2 / den

    a1 = (ids6 == i1).astype(jnp.float32)
    a2 = (ids6 == i2).astype(jnp.float32)
    ri = lax.broadcasted_iota(jnp.int32, (N_TOK, N_TOK), 0)
    ci = lax.broadcasted_iota(jnp.int32, (N_TOK, N_TOK), 1)
    tri = (ci < ri).astype(jnp.float32)
    a12 = jnp.concatenate([a1, a2], axis=1)
    pr = jnp.dot(tri, a12, preferred_element_type=jnp.float32)
    p1, p2r = pr[:, :N_EXPERTS], pr[:, N_EXPERTS:]
    c1 = jnp.sum(a1, axis=0, keepdims=True)
    c2 = jnp.sum(a2, axis=0, keepdims=True)
    rank1 = jnp.sum(a1 * p1, axis=-1, keepdims=True)
    rank2 = jnp.sum(a2 * (p2r + c1), axis=-1, keepdims=True)
    gsz = c1 + c2
    gblk = jnp.floor((gsz + (RB - 1)) * (1.0 / RB))
    e6 = lax.broadcasted_iota(jnp.int32, (N_EXPERTS, N_EXPERTS), 0)
    f6 = lax.broadcasted_iota(jnp.int32, (N_EXPERTS, N_EXPERTS), 1)
    tri6 = (e6 < f6).astype(jnp.float32)
    offblk = jnp.dot(gblk, tri6, preferred_element_type=jnp.float32)
    offrow = offblk * RB
    slot1 = jnp.sum(a1 * offrow, axis=-1, keepdims=True) + rank1
    slot2 = jnp.sum(a2 * offrow, axis=-1, keepdims=True) + rank2
    slots_ref[:N_TOK, :] = slot1.astype(jnp.int32)
    slots_ref[N_TOK:, :] = slot2.astype(jnp.int32)
    bidx = lax.broadcasted_iota(jnp.int32, (NBLK, N_EXPERTS), 0).astype(jnp.float32)
    be_ref[...] = (
        jnp.sum((bidx >= offblk).astype(jnp.int32), axis=-1, keepdims=True) - 1
    )


def _router_tables(x, wr):
    return pl.pallas_call(
        _router_kernel,
        out_shape=(
            jax.ShapeDtypeStruct((N_ASSIGN, 1), jnp.int32),
            jax.ShapeDtypeStruct((N_TOK, 1), jnp.float32),
            jax.ShapeDtypeStruct((N_TOK, 1), jnp.float32),
            jax.ShapeDtypeStruct((NBLK, 1), jnp.int32),
        ),
    )(x, wr)


def _sc_worker_id():
    return lax.axis_index("s") * 2 + lax.axis_index("c")


@functools.lru_cache(maxsize=None)
def _sc_kernels():
    mesh = plsc.VectorSubcoreMesh(core_axis_name="c", subcore_axis_name="s")
    scratch = [
        pltpu.VMEM((SC_CHUNK,), jnp.int32),
        pltpu.VMEM((SC_CHUNK, D_MODEL), jnp.float32),
        pltpu.SemaphoreType.DMA,
    ]

    @functools.partial(
        pl.kernel, mesh=mesh,
        out_type=jax.ShapeDtypeStruct((NSLOT, D_MODEL), jnp.float32),
        scratch_types=scratch,
    )
    def dispatch(x_hbm, slots_hbm, xs_hbm, idx_v, rows_v, sem):
        wid = _sc_worker_id()

        @pl.when(wid < SC_WORKERS)
        def _():
            base = wid * SC_CHUNK
            tok_base = (wid % (SC_WORKERS // 2)) * SC_CHUNK
            pltpu.sync_copy(slots_hbm.at[pl.ds(base, SC_CHUNK)], idx_v)
            pltpu.sync_copy(x_hbm.at[pl.ds(tok_base, SC_CHUNK)], rows_v)
            pltpu.async_copy(rows_v, xs_hbm.at[idx_v], sem).wait()

    @functools.partial(
        pl.kernel, mesh=mesh,
        out_type=jax.ShapeDtypeStruct((N_ASSIGN, D_MODEL), jnp.float32),
        scratch_types=scratch,
    )
    def combine(y_hbm, slots_hbm, yg_hbm, idx_v, rows_v, sem):
        wid = _sc_worker_id()

        @pl.when(wid < SC_WORKERS)
        def _():
            base = wid * SC_CHUNK
            pltpu.sync_copy(slots_hbm.at[pl.ds(base, SC_CHUNK)], idx_v)
            pltpu.async_copy(y_hbm.at[idx_v], rows_v, sem).wait()
            pltpu.sync_copy(rows_v, yg_hbm.at[pl.ds(base, SC_CHUNK)])

    return dispatch, combine


def _sc_dispatch(x, slots):
    return _sc_kernels()[0](x, slots)


def _sc_combine(y, slots):
    return _sc_kernels()[1](y, slots)


def _gffn_kernel(be_ref, xs_ref, wg_ref, bg_ref, wu_ref, bu_ref, wd_ref, bd_ref,
                 y_ref):
    xs = xs_ref[...]
    h = jnp.dot(xs, wg_ref[0], preferred_element_type=jnp.float32) + bg_ref[0]
    u = jnp.dot(xs, wu_ref[0], preferred_element_type=jnp.float32) + bu_ref[0]
    act = jax.nn.silu(h) * u
    y_ref[...] = (
        jnp.dot(act, wd_ref[0], preferred_element_type=jnp.float32) + bd_ref[0]
    )


def _grouped_ffn(be, xs, p):
    grid_spec = pltpu.PrefetchScalarGridSpec(
        num_scalar_prefetch=1,
        grid=(NBLK,),
        in_specs=[
            pl.BlockSpec((RB, D_MODEL), lambda g, be_ref: (g, 0)),
            pl.BlockSpec((1, D_MODEL, D_FF), lambda g, be_ref: (be_ref[g], 0, 0)),
            pl.BlockSpec((1, 1, D_FF), lambda g, be_ref: (be_ref[g], 0, 0)),
            pl.BlockSpec((1, D_MODEL, D_FF), lambda g, be_ref: (be_ref[g], 0, 0)),
            pl.BlockSpec((1, 1, D_FF), lambda g, be_ref: (be_ref[g], 0, 0)),
            pl.BlockSpec((1, D_FF, D_MODEL), lambda g, be_ref: (be_ref[g], 0, 0)),
            pl.BlockSpec((1, 1, D_MODEL), lambda g, be_ref: (be_ref[g], 0, 0)),
        ],
        out_specs=pl.BlockSpec((RB, D_MODEL), lambda g, be_ref: (g, 0)),
    )
    return pl.pallas_call(
        _gffn_kernel,
        grid_spec=grid_spec,
        out_shape=jax.ShapeDtypeStruct((NSLOT, D_MODEL), jnp.float32),
    )(be, xs, p["Wg"], p["bg"][:, None, :], p["Wu"], p["bu"][:, None, :],
      p["Wd"], p["bd"][:, None, :])


def _combine_ln_kernel(x_ref, y1_ref, y2_ref, w1_ref, w2_ref, g_ref, b_ref,
                       o_ref):
    y = x_ref[...] + w1_ref[...] * y1_ref[...] + w2_ref[...] * y2_ref[...]
    o_ref[...] = _ln_op(y, g_ref[...], b_ref[...])


def _moe_ln(x, p, g, gb):
    slots, w1, w2, be = _router_tables(x, p["Wr"])
    slots1d = slots.reshape(N_ASSIGN)
    xs = _sc_dispatch(x, slots1d)
    y = _grouped_ffn(be.reshape(NBLK), xs, p)
    yg = _sc_combine(y, slots1d)
    return pl.pallas_call(
        _combine_ln_kernel,
        out_shape=jax.ShapeDtypeStruct((N_TOK, D_MODEL), jnp.float32),
    )(x, yg[:N_TOK], yg[N_TOK:], w1, w2,
      g.reshape(1, D_MODEL), gb.reshape(1, D_MODEL))


def kernel(x, x_root, x_attr, feature_semantic, feature_key, feature_scene_offset,
           feature_motion, feature_emotion, params):
    bsz, t_chord = x_root.shape
    t_video = feature_scene_offset.shape[1]

    xr = jnp.take(params["emb_root"], x_root, axis=0)
    xa = jnp.take(params["emb_attr"], x_attr, axis=0)
    xe = xr + xa
    fkey = jnp.broadcast_to(feature_key[:, 0][:, None, None], (bsz, t_chord, 1))
    xc = jnp.concatenate([xe, fkey], axis=-1)
    xf = _mm(xc.reshape(bsz * t_chord, D_MODEL + 1), params["Wc"], params["bc"])
    xf = xf.reshape(bsz, t_chord, D_MODEL)

    vf_concat = jnp.concatenate([
        feature_semantic.astype(jnp.float32),
        feature_scene_offset[..., None].astype(jnp.float32),
        feature_motion[..., None].astype(jnp.float32),
        feature_emotion.astype(jnp.float32)], axis=-1)
    vdim = vf_concat.shape[-1]
    vf = _mm(vf_concat.reshape(bsz * t_video, vdim), params["Wv"], params["bv"])
    vf = vf.reshape(bsz, t_video, D_MODEL)

    xf = xf.transpose(1, 0, 2) + params["pos"][:t_chord, None, :]
    vf = vf.transpose(1, 0, 2) + params["pos_video"][:t_video, None, :]
    xf = xf.reshape(t_chord * bsz, D_MODEL)
    vf = vf.reshape(t_video * bsz, D_MODEL)

    zero_mask = jnp.zeros((t_video, t_video), jnp.float32)
    causal_mask = jnp.where(
        jnp.tril(jnp.ones((t_chord, t_chord), dtype=bool)), 0.0, -jnp.inf
    ).astype(jnp.float32)

    h = vf
    for p in params["enc_layers"]:
        a = _mha(h, h, p["attn"], zero_mask, t_video, t_video, bsz)
        h = _mm_res_ln(a, p["attn"]["Wo"], p["attn"]["bo"], h,
                       p["ln1"]["g"], p["ln1"]["b"])
        h = _moe_ln(h, p["moe"], p["ln2"]["g"], p["ln2"]["b"])
    mem = _ln(h, params["enc_norm"]["g"], params["enc_norm"]["b"])

    h = xf
    for p in params["dec_layers"]:
        a = _mha(h, h, p["sattn"], causal_mask, t_chord, t_chord, bsz)
        h = _mm_res_ln(a, p["sattn"]["Wo"], p["sattn"]["bo"], h,
                       p["ln1"]["g"], p["ln1"]["b"])
        a = _mha(h, mem, p["xattn"], zero_mask, t_chord, t_video, bsz)
        h = _mm_res_ln(a, p["xattn"]["Wo"], p["xattn"]["bo"], h,
                       p["ln2"]["g"], p["ln2"]["b"])
        h = _moe_ln(h, p["moe"], p["ln3"]["g"], p["ln3"]["b"])
    out = _ln(h, params["dec_norm"]["g"], params["dec_norm"]["b"])

    out = out.reshape(t_chord, bsz, D_MODEL).transpose(1, 0, 2)
    y = _mm(out.reshape(bsz * t_chord, D_MODEL), params["Wout"], params["bout"])
    return y.reshape(bsz, t_chord, -1)
